# Initial kernel scaffold; baseline (speedup 1.0000x reference)
import jax
import jax.numpy as jnp
from jax.experimental import pallas as pl


def kernel(heatmap, reg, height, dim, rot, vel):
    raise NotImplementedError("write your pallas kernel here")


# speedup vs baseline: 1.0076x; 1.0076x over previous
import functools

import jax
import jax.numpy as jnp
from jax.experimental import pallas as pl
from jax.experimental.pallas import tpu as pltpu

B, C, H, W = 4, 2, 512, 512
K = 500
OUT_SIZE_FACTOR = 4.0
VOXEL = 0.2
PC_LO = -51.2


def _nms_body(x_ref, o_ref):
    x = x_ref[0]
    s = jax.nn.sigmoid(x)
    ninf = jnp.full((1, W), -jnp.inf, s.dtype)
    up = jnp.concatenate([s[1:], ninf], axis=0)
    dn = jnp.concatenate([ninf, s[:-1]], axis=0)
    m = jnp.maximum(jnp.maximum(up, dn), s)
    ninfc = jnp.full((H, 1), -jnp.inf, s.dtype)
    lf = jnp.concatenate([m[:, 1:], ninfc], axis=1)
    rt = jnp.concatenate([ninfc, m[:, :-1]], axis=1)
    m = jnp.maximum(jnp.maximum(lf, rt), m)
    o_ref[0] = jnp.where(m == s, s, 0.0)


def _nms(heatmap):
    hm = heatmap.reshape(B * C, H, W)
    out = pl.pallas_call(
        _nms_body,
        grid=(B * C,),
        in_specs=[pl.BlockSpec((1, H, W), lambda i: (i, 0, 0))],
        out_specs=pl.BlockSpec((1, H, W), lambda i: (i, 0, 0)),
        out_shape=jax.ShapeDtypeStruct((B * C, H, W), jnp.float32),
    )(hm)
    return out.reshape(B, C, H * W)


def _gather_feat_k(feat, ind):
    b, n, c = feat.shape
    idx = jnp.broadcast_to(ind[:, :, None], (b, ind.shape[1], c))
    return jnp.take_along_axis(feat, idx, axis=1)


def _tg(feat, ind):
    b, c, h, w = feat.shape
    f = feat.transpose(0, 2, 3, 1).reshape(b, h * w, c)
    return _gather_feat_k(f, ind)


def kernel(heatmap, reg, height, dim, rot, vel):
    heat = _nms(heatmap)

    topk_scores, topk_inds = jax.lax.top_k(heat, K)
    topk_score, topk_ind = jax.lax.top_k(topk_scores.reshape(B, C * K), K)
    clses = topk_ind // K
    inds = jnp.take_along_axis(topk_inds.reshape(B, C * K), topk_ind, axis=1)
    ys = (inds // W).astype(jnp.float32)
    xs = (inds % W).astype(jnp.float32)

    reg_g = _tg(reg, inds)
    xs = xs[:, :, None] + reg_g[:, :, 0:1]
    ys = ys[:, :, None] + reg_g[:, :, 1:2]
    rot_g = _tg(rot, inds)
    rot_ang = jnp.arctan2(rot_g[:, :, 0:1], rot_g[:, :, 1:2])
    hei_g = _tg(height, inds)
    dim_g = jnp.exp(_tg(dim, inds))
    vel_g = _tg(vel, inds)
    xs = xs * (OUT_SIZE_FACTOR * VOXEL) + PC_LO
    ys = ys * (OUT_SIZE_FACTOR * VOXEL) + PC_LO
    boxes = jnp.concatenate([xs, ys, hei_g, dim_g, rot_ang, vel_g], axis=2)
    labels = clses.astype(jnp.int32)
    mask = (jnp.all(boxes[:, :, :3] >= -1e4, axis=2)
            & jnp.all(boxes[:, :, :3] <= 1e4, axis=2))
    scores = jnp.where(mask, topk_score, 0.0)
    return boxes, scores, labels
